# baseline (speedup 1.0000x reference)
import functools

import jax
import jax.numpy as jnp
from jax import lax
from jax.experimental import pallas as pl
from jax.experimental.pallas import tpu as pltpu
from jax.experimental.pallas import tpu_sc as plsc

N_NODES = 10000
D_FEAT = 128
N_EDGES = 320000

NC = 2
NS = 16
NW = NC * NS
EDGES_PER_W = N_EDGES // NW
CHUNK = 80
NCHUNKS = EDGES_PER_W // CHUNK
L = 16


def _sc_kernel(idx_src_hbm, idx_tgt_hbm, src_hbm, tgt_hbm, out_hbm,
               idx0_v, idx1_v, rows_s0, rows_t0, rows_s1, rows_t1,
               out_v, tr_v, sem0, sem1):
    wid = lax.axis_index("s") * NC + lax.axis_index("c")
    pltpu.sync_copy(idx_src_hbm.at[wid], idx0_v)
    pltpu.sync_copy(idx_tgt_hbm.at[wid], idx1_v)

    rows = ((rows_s0, rows_t0, sem0), (rows_s1, rows_t1, sem1))
    tbase = lax.iota(jnp.int32, L) * L

    def start(c, b):
        rs, rt, sem = rows[b]
        pltpu.async_copy(src_hbm.at[idx0_v.at[c]], rs, sem)
        pltpu.async_copy(tgt_hbm.at[idx1_v.at[c]], rt, sem)

    def wait(c, b):
        rs, rt, sem = rows[b]
        pltpu.make_async_copy(src_hbm.at[idx0_v.at[c]], rs, sem).wait()
        pltpu.make_async_copy(tgt_hbm.at[idx1_v.at[c]], rt, sem).wait()

    def compute(c, b):
        rs, rt, _ = rows[b]

        def group_body(g, gcarry):
            for k in range(L):
                e = g * L + k
                acc = rs[e, pl.ds(0, L)] * rt[e, pl.ds(0, L)]
                for f in range(1, D_FEAT // L):
                    acc = acc + (rs[e, pl.ds(f * L, L)]
                                 * rt[e, pl.ds(f * L, L)])
                tr_v[pl.ds(k * L, L)] = acc
            res = plsc.load_gather(tr_v, [tbase])
            for p in range(1, L):
                res = res + plsc.load_gather(tr_v, [tbase + p])
            out_v[c, pl.ds(g * L, L)] = res
            return gcarry

        lax.fori_loop(0, CHUNK // L, group_body, 0, unroll=False)

    start(0, 0)

    def pair_body(i, carry):
        c = 2 * i
        start(c + 1, 1)
        wait(c, 0)
        compute(c, 0)
        start(c + 2, 0)
        wait(c + 1, 1)
        compute(c + 1, 1)
        return carry

    lax.fori_loop(0, (NCHUNKS - 1) // 2, pair_body, 0, unroll=False)
    wait(NCHUNKS - 1, 0)
    compute(NCHUNKS - 1, 0)

    pltpu.sync_copy(out_v, out_hbm.at[wid])


@jax.jit
def _run(idx_src, idx_tgt, src_emb, tgt_emb):
    mesh = plsc.VectorSubcoreMesh(
        core_axis_name="c", subcore_axis_name="s",
        num_cores=NC, num_subcores=NS)
    kern = pl.kernel(
        _sc_kernel,
        out_type=jax.ShapeDtypeStruct((NW, NCHUNKS, CHUNK), jnp.float32),
        mesh=mesh,
        compiler_params=pltpu.CompilerParams(needs_layout_passes=False),
        scratch_types=[
            pltpu.VMEM((NCHUNKS, CHUNK), jnp.int32),
            pltpu.VMEM((NCHUNKS, CHUNK), jnp.int32),
            pltpu.VMEM((CHUNK, D_FEAT), jnp.float32),
            pltpu.VMEM((CHUNK, D_FEAT), jnp.float32),
            pltpu.VMEM((CHUNK, D_FEAT), jnp.float32),
            pltpu.VMEM((CHUNK, D_FEAT), jnp.float32),
            pltpu.VMEM((NCHUNKS, CHUNK), jnp.float32),
            pltpu.VMEM((L * L,), jnp.float32),
            pltpu.SemaphoreType.DMA,
            pltpu.SemaphoreType.DMA,
        ],
    )
    return kern(idx_src, idx_tgt, src_emb, tgt_emb)


def kernel(source_node_emb, target_node_emb, edge_label_index):
    idx = edge_label_index.astype(jnp.int32).reshape(2, NW, NCHUNKS, CHUNK)
    out = _run(idx[0], idx[1], source_node_emb, target_node_emb)
    return out.reshape(N_EDGES)
